# baseline (speedup 1.0000x reference)
import functools

import jax
import jax.numpy as jnp
from jax import lax
from jax.experimental import pallas as pl
from jax.experimental.pallas import tpu as pltpu
from jax.experimental.pallas import tpu_sc as plsc

N = 50000
E = 800000
NP = 51200
EP = 819200
NW = 32
CH = 128
CPT = EP // NW // CH
NCH = EP // CH
EPT = EP // NW
RPS = NP // 16
DUMMY = 50100
NB = 8
NGRP = CPT // NB
BLK = 1024
GRID_N = NP // BLK
SHIFT = 848
EPS = 821248
NV = N + E
GRID_V = (NV + BLK - 1) // BLK

_mesh = plsc.VectorSubcoreMesh(core_axis_name="c", subcore_axis_name="s")
_sc_params = pltpu.CompilerParams(use_tc_tiling_on_sc=False)


def _sc_scatter(w):
  @functools.partial(
      pl.kernel,
      out_type=jax.ShapeDtypeStruct((2, NP, w), jnp.float32),
      mesh=_mesh,
      compiler_params=_sc_params,
      scratch_types=[
          pltpu.VMEM((CPT, CH), jnp.int32),
          pltpu.VMEM((CPT, CH), jnp.int32),
          pltpu.VMEM((NB, CH, w), jnp.float32),
          pltpu.VMEM_SHARED((NP, w), jnp.float32),
          pltpu.SemaphoreType.DMA((NB,)),
          pltpu.SemaphoreType.DMA((NB,)),
      ],
  )
  def k(src_hbm, dst_hbm, y_hbm, z_hbm, out_hbm, sidx, didx, rows, acc_sh,
        gsem, ssem):
    c = lax.axis_index("c")
    s = lax.axis_index("s")
    wid = s * 2 + c
    pltpu.sync_copy(z_hbm.at[pl.ds(s * RPS, RPS)], acc_sh.at[pl.ds(s * RPS, RPS)])
    pltpu.sync_copy(src_hbm.at[pl.ds(wid * CPT, CPT)], sidx)
    pltpu.sync_copy(dst_hbm.at[pl.ds(wid * CPT, CPT)], didx)
    plsc.subcore_barrier()

    for b in range(NB):
      pltpu.async_copy(y_hbm.at[sidx.at[b]], rows.at[b], gsem.at[b])

    def outer(g, carry):
      base = g * NB
      for b in range(NB):
        pltpu.make_async_copy(y_hbm.at[sidx.at[base + b]], rows.at[b],
                              gsem.at[b]).wait()
        pltpu.async_copy(rows.at[b], acc_sh.at[didx.at[base + b]], ssem.at[b],
                         add=True)
      for b in range(NB):
        nxt = base + NB + b

        @pl.when(nxt < CPT)
        def _():
          pltpu.make_async_copy(rows.at[b], acc_sh.at[didx.at[base + b]],
                                ssem.at[b]).wait()
          pltpu.async_copy(y_hbm.at[sidx.at[nxt]], rows.at[b], gsem.at[b])

      return carry

    lax.fori_loop(0, NGRP, outer, 0)
    for b in range(NB):
      pltpu.make_async_copy(rows.at[b], acc_sh.at[didx.at[CPT - NB + b]],
                            ssem.at[b]).wait()
    plsc.subcore_barrier()
    pltpu.sync_copy(acc_sh.at[pl.ds(s * RPS, RPS)],
                    out_hbm.at[c].at[pl.ds(s * RPS, RPS)])

  return k


@functools.partial(
    pl.kernel,
    out_type=[jax.ShapeDtypeStruct((2, NP, 16), jnp.float32),
              jax.ShapeDtypeStruct((2, NP, 16), jnp.float32)],
    mesh=_mesh,
    compiler_params=_sc_params,
    scratch_types=[
        pltpu.VMEM((CPT, CH), jnp.int32),
        pltpu.VMEM((CPT, CH), jnp.int32),
        pltpu.VMEM((NB, CH, 16), jnp.float32),
        pltpu.VMEM_SHARED((NP, 16), jnp.float32),
        pltpu.SemaphoreType.DMA((NB,)),
        pltpu.SemaphoreType.DMA((NB,)),
    ],
)
def _sc_scatter16x2(src_hbm, dst_hbm, ya_hbm, yb_hbm, z_hbm, outa_hbm, outb_hbm,
                    sidx, didx, rows, acc_sh, gsem, ssem):
  c = lax.axis_index("c")
  s = lax.axis_index("s")
  wid = s * 2 + c
  pltpu.sync_copy(src_hbm.at[pl.ds(wid * CPT, CPT)], sidx)
  pltpu.sync_copy(dst_hbm.at[pl.ds(wid * CPT, CPT)], didx)

  for y_hbm, out_hbm in ((ya_hbm, outa_hbm), (yb_hbm, outb_hbm)):
    pltpu.sync_copy(z_hbm.at[pl.ds(s * RPS, RPS)], acc_sh.at[pl.ds(s * RPS, RPS)])
    plsc.subcore_barrier()

    for b in range(NB):
      pltpu.async_copy(y_hbm.at[sidx.at[b]], rows.at[b], gsem.at[b])

    def outer(g, carry):
      base = g * NB
      for b in range(NB):
        pltpu.make_async_copy(y_hbm.at[sidx.at[base + b]], rows.at[b],
                              gsem.at[b]).wait()
        pltpu.async_copy(rows.at[b], acc_sh.at[didx.at[base + b]], ssem.at[b],
                         add=True)
      for b in range(NB):
        nxt = base + NB + b

        @pl.when(nxt < CPT)
        def _():
          pltpu.make_async_copy(rows.at[b], acc_sh.at[didx.at[base + b]],
                                ssem.at[b]).wait()
          pltpu.async_copy(y_hbm.at[sidx.at[nxt]], rows.at[b], gsem.at[b])

      return carry

    lax.fori_loop(0, NGRP, outer, 0)
    for b in range(NB):
      pltpu.make_async_copy(rows.at[b], acc_sh.at[didx.at[CPT - NB + b]],
                            ssem.at[b]).wait()
    plsc.subcore_barrier()
    pltpu.sync_copy(acc_sh.at[pl.ds(s * RPS, RPS)],
                    out_hbm.at[c].at[pl.ds(s * RPS, RPS)])
    plsc.subcore_barrier()


@functools.partial(
    pl.kernel,
    out_type=jax.ShapeDtypeStruct((2, NP), jnp.float32),
    mesh=_mesh,
    compiler_params=_sc_params,
    scratch_types=[
        pltpu.VMEM((CPT, CH), jnp.int32),
        pltpu.VMEM((CH,), jnp.float32),
        pltpu.VMEM_SHARED((NP,), jnp.float32),
        pltpu.SemaphoreType.DMA((NB,)),
    ],
)
def _sc_degree(dst_hbm, z_hbm, out_hbm, didx, ones_v, acc_sh, ssem):
  c = lax.axis_index("c")
  s = lax.axis_index("s")
  wid = s * 2 + c
  one = jnp.full((16,), 1.0, jnp.float32)
  for j in range(CH // 16):
    ones_v[pl.ds(j * 16, 16)] = one
  pltpu.sync_copy(z_hbm.at[pl.ds(s * RPS, RPS)], acc_sh.at[pl.ds(s * RPS, RPS)])
  pltpu.sync_copy(dst_hbm.at[pl.ds(wid * CPT, CPT)], didx)
  plsc.subcore_barrier()

  for b in range(NB):
    pltpu.async_copy(ones_v, acc_sh.at[didx.at[b]], ssem.at[b], add=True)

  def outer(g, carry):
    base = g * NB
    for b in range(NB):
      nxt = base + NB + b

      @pl.when(nxt < CPT)
      def _():
        pltpu.make_async_copy(ones_v, acc_sh.at[didx.at[base + b]],
                              ssem.at[b]).wait()
        pltpu.async_copy(ones_v, acc_sh.at[didx.at[nxt]], ssem.at[b], add=True)

    return carry

  lax.fori_loop(0, NGRP, outer, 0)
  for b in range(NB):
    pltpu.make_async_copy(ones_v, acc_sh.at[didx.at[CPT - NB + b]],
                          ssem.at[b]).wait()
  plsc.subcore_barrier()
  pltpu.sync_copy(acc_sh.at[pl.ds(s * RPS, RPS)],
                  out_hbm.at[c].at[pl.ds(s * RPS, RPS)])


@functools.partial(
    pl.kernel,
    out_type=[jax.ShapeDtypeStruct((EPS, 8), jnp.float32),
              jax.ShapeDtypeStruct((EPS, 8), jnp.float32)],
    mesh=_mesh,
    compiler_params=_sc_params,
    scratch_types=[
        pltpu.VMEM((CPT, CH), jnp.int32),
        pltpu.VMEM((NB, CH, 8), jnp.float32),
        pltpu.SemaphoreType.DMA((NB,)),
        pltpu.SemaphoreType.DMA((NB,)),
    ],
)
def _sc_gather2(src_hbm, dst_hbm, vh_hbm, gs_hbm, gd_hbm, sidx, rows,
                gsem, wsem):
  c = lax.axis_index("c")
  s = lax.axis_index("s")
  wid = s * 2 + c
  base0 = SHIFT + wid * EPT

  for idx_hbm, out_hbm in ((src_hbm, gs_hbm), (dst_hbm, gd_hbm)):
    pltpu.sync_copy(idx_hbm.at[pl.ds(wid * CPT, CPT)], sidx)

    for b in range(NB):
      pltpu.async_copy(vh_hbm.at[sidx.at[b]], rows.at[b], gsem.at[b])

    def outer(g, carry):
      base = g * NB
      for b in range(NB):
        ob = base0 + (base + b) * CH
        pltpu.make_async_copy(vh_hbm.at[sidx.at[base + b]], rows.at[b],
                              gsem.at[b]).wait()
        pltpu.async_copy(rows.at[b], out_hbm.at[pl.ds(ob, CH)], wsem.at[b])
      for b in range(NB):
        nxt = base + NB + b
        ob = base0 + (base + b) * CH

        @pl.when(nxt < CPT)
        def _():
          pltpu.make_async_copy(rows.at[b], out_hbm.at[pl.ds(ob, CH)],
                                wsem.at[b]).wait()
          pltpu.async_copy(vh_hbm.at[sidx.at[nxt]], rows.at[b], gsem.at[b])

      return carry

    lax.fori_loop(0, NGRP, outer, 0)
    for b in range(NB):
      obl = base0 + (CPT - NB + b) * CH
      pltpu.make_async_copy(rows.at[b], out_hbm.at[pl.ds(obl, CH)],
                            wsem.at[b]).wait()


def _leaky(x):
  return jnp.where(x >= 0, x, 0.01 * x)


def _row_spec(w):
  if w == 1:
    return pl.BlockSpec((BLK,), lambda i: (i,))
  return pl.BlockSpec((BLK, w), lambda i: (i, 0))


def _full_spec(shape):
  nd = len(shape)
  return pl.BlockSpec(shape, lambda i: (0,) * nd)


def _part_spec(w, which):
  if w == 1:
    return pl.BlockSpec((1, BLK), lambda i: (which, i))
  return pl.BlockSpec((1, BLK, w), lambda i: (which, i, 0))


def _tc0_body(d0, d1, v8, dis, y1):
  deg = d0[...] + d1[...] + 1.0
  r = lax.rsqrt(deg)
  dis[...] = r
  y1[...] = v8[...] * r[:, None]


def _tc1_body(a0, a1, y1, dis, w1, b1, w2, y2a, y2b):
  t = (a0[...][0] + a1[...][0] + y1[...]) * dis[...][:, None]
  x1 = _leaky(jnp.dot(t[:, :3], w1[...], preferred_element_type=jnp.float32)
              + b1[...][None, :])
  h1 = jnp.dot(x1, w2[...], preferred_element_type=jnp.float32)
  y2 = h1 * dis[...][:, None]
  y2a[...] = y2[:, :16]
  y2b[...] = y2[:, 16:]


def _tc2_body(a0a, a1a, a0b, a1b, y2a, y2b, dis, b2, w3a, w3b, y3p):
  d = dis[...][:, None]
  x2a = _leaky((a0a[...][0] + a1a[...][0] + y2a[...]) * d + b2[...][None, :16])
  x2b = _leaky((a0b[...][0] + a1b[...][0] + y2b[...]) * d + b2[...][None, 16:])
  h2 = (jnp.dot(x2a, w3a[...], preferred_element_type=jnp.float32)
        + jnp.dot(x2b, w3b[...], preferred_element_type=jnp.float32))
  y3p[...] = h2 * d


def _tc3_body(a0, a1, y3p, dis, v8, b3p, vout, vh):
  off = (a0[...][0] + a1[...][0] + y3p[...]) * dis[...][:, None] + b3p[...][None, :]
  v = v8[...] + off
  vout[...] = v
  vh[...] = 0.5 * v


def _tcprep_body(e_ref, src_ref, dst_ref):
  gid = pl.program_id(0)
  rows = gid * BLK + lax.broadcasted_iota(jnp.int32, (BLK,), 0)
  valid = rows < E
  e = e_ref[...]
  src_ref[...] = jnp.where(valid, e[:, 0], 0)
  dst_ref[...] = jnp.where(valid, e[:, 1], DUMMY)


def _tcasm_body(v8, gs, gd, out):
  gid = pl.program_id(0)
  rows2 = gid * BLK + lax.broadcasted_iota(jnp.int32, (BLK, 3), 0)
  mid = gs[...] + gd[...]
  out[...] = jnp.where(rows2 < N, v8[...][:, :3], mid[:, :3])[None]


_scatter8 = _sc_scatter(8)
_scatter16 = _sc_scatter(16)

_tc0 = pl.pallas_call(
    _tc0_body, grid=(GRID_N,),
    in_specs=[pl.BlockSpec((BLK,), lambda i: (i,)),
              pl.BlockSpec((BLK,), lambda i: (i + GRID_N,)), _row_spec(8)],
    out_specs=[_row_spec(1), _row_spec(8)],
    out_shape=[jax.ShapeDtypeStruct((NP,), jnp.float32),
               jax.ShapeDtypeStruct((NP, 8), jnp.float32)],
)

_tc1 = pl.pallas_call(
    _tc1_body, grid=(GRID_N,),
    in_specs=[_part_spec(8, 0), _part_spec(8, 1), _row_spec(8), _row_spec(1),
              _full_spec((3, 64)), _full_spec((64,)), _full_spec((64, 32))],
    out_specs=[_row_spec(16), _row_spec(16)],
    out_shape=[jax.ShapeDtypeStruct((NP, 16), jnp.float32),
               jax.ShapeDtypeStruct((NP, 16), jnp.float32)],
)

_tc2 = pl.pallas_call(
    _tc2_body, grid=(GRID_N,),
    in_specs=[_part_spec(16, 0), _part_spec(16, 1), _part_spec(16, 0),
              _part_spec(16, 1), _row_spec(16), _row_spec(16), _row_spec(1),
              _full_spec((32,)), _full_spec((16, 8)), _full_spec((16, 8))],
    out_specs=_row_spec(8),
    out_shape=jax.ShapeDtypeStruct((NP, 8), jnp.float32),
)

_tc3 = pl.pallas_call(
    _tc3_body, grid=(GRID_N,),
    in_specs=[_part_spec(8, 0), _part_spec(8, 1), _row_spec(8), _row_spec(1),
              _row_spec(8), _full_spec((8,))],
    out_specs=[_row_spec(8), _row_spec(8)],
    out_shape=[jax.ShapeDtypeStruct((NP, 8), jnp.float32),
               jax.ShapeDtypeStruct((NP, 8), jnp.float32)],
)

_tcprep = pl.pallas_call(
    _tcprep_body, grid=(EP // BLK,),
    in_specs=[pl.BlockSpec((BLK, 2), lambda i: (jnp.minimum(i, E // BLK), 0))],
    out_specs=[_row_spec(1), _row_spec(1)],
    out_shape=[jax.ShapeDtypeStruct((EP,), jnp.int32),
               jax.ShapeDtypeStruct((EP,), jnp.int32)],
)

_tcasm = pl.pallas_call(
    _tcasm_body, grid=(GRID_V,),
    in_specs=[pl.BlockSpec((BLK, 8), lambda i: (jnp.minimum(i, GRID_N - 1), 0)),
              pl.BlockSpec((BLK, 8), lambda i: (jnp.maximum(i - 48, 0), 0)),
              pl.BlockSpec((BLK, 8), lambda i: (jnp.maximum(i - 48, 0), 0))],
    out_specs=pl.BlockSpec((1, BLK, 3), lambda i: (0, i, 0)),
    out_shape=jax.ShapeDtypeStruct((1, NV, 3), jnp.float32),
)


def kernel(verts, edges, subdivided_faces, W1, b1, W2, b2, W3, b3):
  srcp, dstp = _tcprep(edges)
  src2 = srcp.reshape(NCH, CH)
  dst2 = dstp.reshape(NCH, CH)
  verts8 = jnp.pad(verts, ((0, NP - N), (0, 5)))
  w3p = jnp.pad(W3, ((0, 0), (0, 5)))
  w3a = w3p[:16]
  w3b = w3p[16:]
  b3p = jnp.pad(b3, (0, 5))
  z1 = jnp.zeros((NP,), jnp.float32)
  z8 = jnp.zeros((NP, 8), jnp.float32)
  z16 = jnp.zeros((NP, 16), jnp.float32)

  degf = _sc_degree(dst2, z1).reshape(2 * NP)
  dis, y1 = _tc0(degf, degf, verts8)
  acc1 = _scatter8(src2, dst2, y1, z8)
  y2a, y2b = _tc1(acc1, acc1, y1, dis, W1, b1, W2)
  acc2a, acc2b = _sc_scatter16x2(src2, dst2, y2a, y2b, z16)
  y3p = _tc2(acc2a, acc2a, acc2b, acc2b, y2a, y2b, dis, b2, w3a, w3b)
  acc3 = _scatter8(src2, dst2, y3p, z8)
  v8, vh = _tc3(acc3, acc3, y3p, dis, verts8, b3p)
  gs, gd = _sc_gather2(src2, dst2, vh)
  new_verts = _tcasm(v8, gs, gd)
  new_faces = subdivided_faces[None]
  return new_verts, new_faces


# speedup vs baseline: 1.0375x; 1.0375x over previous
import functools

import jax
import jax.numpy as jnp
from jax import lax
from jax.experimental import pallas as pl
from jax.experimental.pallas import tpu as pltpu
from jax.experimental.pallas import tpu_sc as plsc

N = 50000
E = 800000
NP = 51200
EP = 819200
NW = 32
CH = 128
CPT = EP // NW // CH
NCH = EP // CH
EPT = EP // NW
RPS = NP // 16
DUMMY = 50100
NB = 8
NGRP = CPT // NB
BLK = 1024
GRID_N = NP // BLK
SHIFT = 848
EPS = 821248
NV = N + E
GRID_V = (NV + BLK - 1) // BLK

_mesh = plsc.VectorSubcoreMesh(core_axis_name="c", subcore_axis_name="s")
_sc_params = pltpu.CompilerParams(use_tc_tiling_on_sc=False)


def _sc_scatter(w):
  @functools.partial(
      pl.kernel,
      out_type=jax.ShapeDtypeStruct((2, NP, w), jnp.float32),
      mesh=_mesh,
      compiler_params=_sc_params,
      scratch_types=[
          pltpu.VMEM((CPT, CH), jnp.int32),
          pltpu.VMEM((CPT, CH), jnp.int32),
          pltpu.VMEM((NB, CH, w), jnp.float32),
          pltpu.VMEM_SHARED((NP, w), jnp.float32),
          pltpu.SemaphoreType.DMA((NB,)),
          pltpu.SemaphoreType.DMA((NB,)),
      ],
  )
  def k(src_hbm, dst_hbm, y_hbm, z_hbm, out_hbm, sidx, didx, rows, acc_sh,
        gsem, ssem):
    c = lax.axis_index("c")
    s = lax.axis_index("s")
    wid = s * 2 + c
    pltpu.sync_copy(z_hbm.at[pl.ds(s * RPS, RPS)], acc_sh.at[pl.ds(s * RPS, RPS)])
    pltpu.sync_copy(src_hbm.at[pl.ds(wid * CPT, CPT)], sidx)
    pltpu.sync_copy(dst_hbm.at[pl.ds(wid * CPT, CPT)], didx)
    plsc.subcore_barrier()

    for b in range(NB):
      pltpu.async_copy(y_hbm.at[sidx.at[b]], rows.at[b], gsem.at[b])

    def outer(g, carry):
      base = g * NB
      for b in range(NB):
        pltpu.make_async_copy(y_hbm.at[sidx.at[base + b]], rows.at[b],
                              gsem.at[b]).wait()
        pltpu.async_copy(rows.at[b], acc_sh.at[didx.at[base + b]], ssem.at[b],
                         add=True)
      for b in range(NB):
        nxt = base + NB + b

        @pl.when(nxt < CPT)
        def _():
          pltpu.make_async_copy(rows.at[b], acc_sh.at[didx.at[base + b]],
                                ssem.at[b]).wait()
          pltpu.async_copy(y_hbm.at[sidx.at[nxt]], rows.at[b], gsem.at[b])

      return carry

    lax.fori_loop(0, NGRP, outer, 0)
    for b in range(NB):
      pltpu.make_async_copy(rows.at[b], acc_sh.at[didx.at[CPT - NB + b]],
                            ssem.at[b]).wait()
    plsc.subcore_barrier()
    pltpu.sync_copy(acc_sh.at[pl.ds(s * RPS, RPS)],
                    out_hbm.at[c].at[pl.ds(s * RPS, RPS)])

  return k


@functools.partial(
    pl.kernel,
    out_type=[jax.ShapeDtypeStruct((2, NP, 16), jnp.float32),
              jax.ShapeDtypeStruct((2, NP, 16), jnp.float32)],
    mesh=_mesh,
    compiler_params=_sc_params,
    scratch_types=[
        pltpu.VMEM((CPT, CH), jnp.int32),
        pltpu.VMEM((CPT, CH), jnp.int32),
        pltpu.VMEM((NB, CH, 16), jnp.float32),
        pltpu.VMEM_SHARED((NP, 16), jnp.float32),
        pltpu.SemaphoreType.DMA((NB,)),
        pltpu.SemaphoreType.DMA((NB,)),
    ],
)
def _sc_scatter16x2(src_hbm, dst_hbm, ya_hbm, yb_hbm, z_hbm, outa_hbm, outb_hbm,
                    sidx, didx, rows, acc_sh, gsem, ssem):
  c = lax.axis_index("c")
  s = lax.axis_index("s")
  wid = s * 2 + c
  pltpu.sync_copy(src_hbm.at[pl.ds(wid * CPT, CPT)], sidx)
  pltpu.sync_copy(dst_hbm.at[pl.ds(wid * CPT, CPT)], didx)

  for y_hbm, out_hbm in ((ya_hbm, outa_hbm), (yb_hbm, outb_hbm)):
    pltpu.sync_copy(z_hbm.at[pl.ds(s * RPS, RPS)], acc_sh.at[pl.ds(s * RPS, RPS)])
    plsc.subcore_barrier()

    for b in range(NB):
      pltpu.async_copy(y_hbm.at[sidx.at[b]], rows.at[b], gsem.at[b])

    def outer(g, carry):
      base = g * NB
      for b in range(NB):
        pltpu.make_async_copy(y_hbm.at[sidx.at[base + b]], rows.at[b],
                              gsem.at[b]).wait()
        pltpu.async_copy(rows.at[b], acc_sh.at[didx.at[base + b]], ssem.at[b],
                         add=True)
      for b in range(NB):
        nxt = base + NB + b

        @pl.when(nxt < CPT)
        def _():
          pltpu.make_async_copy(rows.at[b], acc_sh.at[didx.at[base + b]],
                                ssem.at[b]).wait()
          pltpu.async_copy(y_hbm.at[sidx.at[nxt]], rows.at[b], gsem.at[b])

      return carry

    lax.fori_loop(0, NGRP, outer, 0)
    for b in range(NB):
      pltpu.make_async_copy(rows.at[b], acc_sh.at[didx.at[CPT - NB + b]],
                            ssem.at[b]).wait()
    plsc.subcore_barrier()
    pltpu.sync_copy(acc_sh.at[pl.ds(s * RPS, RPS)],
                    out_hbm.at[c].at[pl.ds(s * RPS, RPS)])
    plsc.subcore_barrier()


@functools.partial(
    pl.kernel,
    out_type=jax.ShapeDtypeStruct((2, NP), jnp.float32),
    mesh=_mesh,
    compiler_params=_sc_params,
    scratch_types=[
        pltpu.VMEM((CPT, CH), jnp.int32),
        pltpu.VMEM((CH,), jnp.float32),
        pltpu.VMEM_SHARED((NP,), jnp.float32),
        pltpu.SemaphoreType.DMA((NB,)),
    ],
)
def _sc_degree(dst_hbm, z_hbm, out_hbm, didx, ones_v, acc_sh, ssem):
  c = lax.axis_index("c")
  s = lax.axis_index("s")
  wid = s * 2 + c
  one = jnp.full((16,), 1.0, jnp.float32)
  for j in range(CH // 16):
    ones_v[pl.ds(j * 16, 16)] = one
  pltpu.sync_copy(z_hbm.at[pl.ds(s * RPS, RPS)], acc_sh.at[pl.ds(s * RPS, RPS)])
  pltpu.sync_copy(dst_hbm.at[pl.ds(wid * CPT, CPT)], didx)
  plsc.subcore_barrier()

  for b in range(NB):
    pltpu.async_copy(ones_v, acc_sh.at[didx.at[b]], ssem.at[b], add=True)

  def outer(g, carry):
    base = g * NB
    for b in range(NB):
      nxt = base + NB + b

      @pl.when(nxt < CPT)
      def _():
        pltpu.make_async_copy(ones_v, acc_sh.at[didx.at[base + b]],
                              ssem.at[b]).wait()
        pltpu.async_copy(ones_v, acc_sh.at[didx.at[nxt]], ssem.at[b], add=True)

    return carry

  lax.fori_loop(0, NGRP, outer, 0)
  for b in range(NB):
    pltpu.make_async_copy(ones_v, acc_sh.at[didx.at[CPT - NB + b]],
                          ssem.at[b]).wait()
  plsc.subcore_barrier()
  pltpu.sync_copy(acc_sh.at[pl.ds(s * RPS, RPS)],
                  out_hbm.at[c].at[pl.ds(s * RPS, RPS)])


@functools.partial(
    pl.kernel,
    out_type=jax.ShapeDtypeStruct((EPS, 8), jnp.float32),
    mesh=_mesh,
    compiler_params=_sc_params,
    scratch_types=[
        pltpu.VMEM((CPT, CH), jnp.int32),
        pltpu.VMEM((NB, CH, 8), jnp.float32),
        pltpu.SemaphoreType.DMA((NB,)),
        pltpu.SemaphoreType.DMA((NB,)),
    ],
)
def _sc_gatherrows(idx_hbm, vh_hbm, out_hbm, sidx, rows, gsem, wsem):
  c = lax.axis_index("c")
  s = lax.axis_index("s")
  wid = s * 2 + c
  base0 = SHIFT + wid * EPT
  pltpu.sync_copy(idx_hbm.at[pl.ds(wid * CPT, CPT)], sidx)

  for b in range(NB):
    pltpu.async_copy(vh_hbm.at[sidx.at[b]], rows.at[b], gsem.at[b])

  def outer(g, carry):
    base = g * NB
    for b in range(NB):
      ob = base0 + (base + b) * CH
      pltpu.make_async_copy(vh_hbm.at[sidx.at[base + b]], rows.at[b],
                            gsem.at[b]).wait()
      pltpu.async_copy(rows.at[b], out_hbm.at[pl.ds(ob, CH)], wsem.at[b])
    for b in range(NB):
      nxt = base + NB + b
      ob = base0 + (base + b) * CH

      @pl.when(nxt < CPT)
      def _():
        pltpu.make_async_copy(rows.at[b], out_hbm.at[pl.ds(ob, CH)],
                              wsem.at[b]).wait()
        pltpu.async_copy(vh_hbm.at[sidx.at[nxt]], rows.at[b], gsem.at[b])

    return carry

  lax.fori_loop(0, NGRP, outer, 0)
  for b in range(NB):
    obl = base0 + (CPT - NB + b) * CH
    pltpu.make_async_copy(rows.at[b], out_hbm.at[pl.ds(obl, CH)],
                          wsem.at[b]).wait()


@functools.partial(
    pl.kernel,
    out_type=[jax.ShapeDtypeStruct((EPS, 8), jnp.float32),
              jax.ShapeDtypeStruct((EPS, 8), jnp.float32)],
    mesh=_mesh,
    compiler_params=_sc_params,
    scratch_types=[
        pltpu.VMEM((CPT, CH), jnp.int32),
        pltpu.VMEM((NB, CH, 8), jnp.float32),
        pltpu.SemaphoreType.DMA((NB,)),
        pltpu.SemaphoreType.DMA((NB,)),
    ],
)
def _sc_gather2(src_hbm, dst_hbm, vh_hbm, gs_hbm, gd_hbm, sidx, rows,
                gsem, wsem):
  c = lax.axis_index("c")
  s = lax.axis_index("s")
  wid = s * 2 + c
  base0 = SHIFT + wid * EPT

  for idx_hbm, out_hbm in ((src_hbm, gs_hbm), (dst_hbm, gd_hbm)):
    pltpu.sync_copy(idx_hbm.at[pl.ds(wid * CPT, CPT)], sidx)

    for b in range(NB):
      pltpu.async_copy(vh_hbm.at[sidx.at[b]], rows.at[b], gsem.at[b])

    def outer(g, carry):
      base = g * NB
      for b in range(NB):
        ob = base0 + (base + b) * CH
        pltpu.make_async_copy(vh_hbm.at[sidx.at[base + b]], rows.at[b],
                              gsem.at[b]).wait()
        pltpu.async_copy(rows.at[b], out_hbm.at[pl.ds(ob, CH)], wsem.at[b])
      for b in range(NB):
        nxt = base + NB + b
        ob = base0 + (base + b) * CH

        @pl.when(nxt < CPT)
        def _():
          pltpu.make_async_copy(rows.at[b], out_hbm.at[pl.ds(ob, CH)],
                                wsem.at[b]).wait()
          pltpu.async_copy(vh_hbm.at[sidx.at[nxt]], rows.at[b], gsem.at[b])

      return carry

    lax.fori_loop(0, NGRP, outer, 0)
    for b in range(NB):
      obl = base0 + (CPT - NB + b) * CH
      pltpu.make_async_copy(rows.at[b], out_hbm.at[pl.ds(obl, CH)],
                            wsem.at[b]).wait()


def _leaky(x):
  return jnp.where(x >= 0, x, 0.01 * x)


def _row_spec(w):
  if w == 1:
    return pl.BlockSpec((BLK,), lambda i: (i,))
  return pl.BlockSpec((BLK, w), lambda i: (i, 0))


def _full_spec(shape):
  nd = len(shape)
  return pl.BlockSpec(shape, lambda i: (0,) * nd)


def _part_spec(w, which):
  if w == 1:
    return pl.BlockSpec((1, BLK), lambda i: (which, i))
  return pl.BlockSpec((1, BLK, w), lambda i: (which, i, 0))


def _tc0_body(d0, d1, v8, dis, y1):
  deg = d0[...] + d1[...] + 1.0
  r = lax.rsqrt(deg)
  dis[...] = r
  y1[...] = v8[...] * r[:, None]


def _tc1_body(a0, a1, y1, dis, w1, b1, w2, y2a, y2b):
  t = (a0[...][0] + a1[...][0] + y1[...]) * dis[...][:, None]
  x1 = _leaky(jnp.dot(t[:, :3], w1[...], preferred_element_type=jnp.float32)
              + b1[...][None, :])
  h1 = jnp.dot(x1, w2[...], preferred_element_type=jnp.float32)
  y2 = h1 * dis[...][:, None]
  y2a[...] = y2[:, :16]
  y2b[...] = y2[:, 16:]


def _tc2_body(a0a, a1a, a0b, a1b, y2a, y2b, dis, b2, w3a, w3b, y3p):
  d = dis[...][:, None]
  x2a = _leaky((a0a[...][0] + a1a[...][0] + y2a[...]) * d + b2[...][None, :16])
  x2b = _leaky((a0b[...][0] + a1b[...][0] + y2b[...]) * d + b2[...][None, 16:])
  h2 = (jnp.dot(x2a, w3a[...], preferred_element_type=jnp.float32)
        + jnp.dot(x2b, w3b[...], preferred_element_type=jnp.float32))
  y3p[...] = h2 * d


def _tc3_body(a0, a1, y3p, dis, v8, b3p, vout, vh):
  off = (a0[...][0] + a1[...][0] + y3p[...]) * dis[...][:, None] + b3p[...][None, :]
  v = v8[...] + off
  vout[...] = v
  vh[...] = 0.5 * v


def _tcprep_body(e_ref, src_ref, dst_ref):
  gid = pl.program_id(0)
  rows = gid * BLK + lax.broadcasted_iota(jnp.int32, (BLK,), 0)
  valid = rows < E
  e = e_ref[...]
  src_ref[...] = jnp.where(valid, e[:, 0], 0)
  dst_ref[...] = jnp.where(valid, e[:, 1], DUMMY)


def _tcasm_body(v8, gs, gd, out):
  gid = pl.program_id(0)
  rows2 = gid * BLK + lax.broadcasted_iota(jnp.int32, (BLK, 3), 0)
  mid = gs[...] + gd[...]
  out[...] = jnp.where(rows2 < N, v8[...][:, :3], mid[:, :3])[None]


_scatter8 = _sc_scatter(8)
_scatter16 = _sc_scatter(16)

_tc0 = pl.pallas_call(
    _tc0_body, grid=(GRID_N,),
    in_specs=[pl.BlockSpec((BLK,), lambda i: (i,)),
              pl.BlockSpec((BLK,), lambda i: (i + GRID_N,)), _row_spec(8)],
    out_specs=[_row_spec(1), _row_spec(8)],
    out_shape=[jax.ShapeDtypeStruct((NP,), jnp.float32),
               jax.ShapeDtypeStruct((NP, 8), jnp.float32)],
)

_tc1 = pl.pallas_call(
    _tc1_body, grid=(GRID_N,),
    in_specs=[_part_spec(8, 0), _part_spec(8, 1), _row_spec(8), _row_spec(1),
              _full_spec((3, 64)), _full_spec((64,)), _full_spec((64, 32))],
    out_specs=[_row_spec(16), _row_spec(16)],
    out_shape=[jax.ShapeDtypeStruct((NP, 16), jnp.float32),
               jax.ShapeDtypeStruct((NP, 16), jnp.float32)],
)

_tc2 = pl.pallas_call(
    _tc2_body, grid=(GRID_N,),
    in_specs=[_part_spec(16, 0), _part_spec(16, 1), _part_spec(16, 0),
              _part_spec(16, 1), _row_spec(16), _row_spec(16), _row_spec(1),
              _full_spec((32,)), _full_spec((16, 8)), _full_spec((16, 8))],
    out_specs=_row_spec(8),
    out_shape=jax.ShapeDtypeStruct((NP, 8), jnp.float32),
)

_tc3 = pl.pallas_call(
    _tc3_body, grid=(GRID_N,),
    in_specs=[_part_spec(8, 0), _part_spec(8, 1), _row_spec(8), _row_spec(1),
              _row_spec(8), _full_spec((8,))],
    out_specs=[_row_spec(8), _row_spec(8)],
    out_shape=[jax.ShapeDtypeStruct((NP, 8), jnp.float32),
               jax.ShapeDtypeStruct((NP, 8), jnp.float32)],
)

_tcprep = pl.pallas_call(
    _tcprep_body, grid=(EP // BLK,),
    in_specs=[pl.BlockSpec((BLK, 2), lambda i: (jnp.minimum(i, E // BLK), 0))],
    out_specs=[_row_spec(1), _row_spec(1)],
    out_shape=[jax.ShapeDtypeStruct((EP,), jnp.int32),
               jax.ShapeDtypeStruct((EP,), jnp.int32)],
)

_tcasm = pl.pallas_call(
    _tcasm_body, grid=(GRID_V,),
    in_specs=[pl.BlockSpec((BLK, 8), lambda i: (jnp.minimum(i, GRID_N - 1), 0)),
              pl.BlockSpec((BLK, 8), lambda i: (jnp.maximum(i - 48, 0), 0)),
              pl.BlockSpec((BLK, 8), lambda i: (jnp.maximum(i - 48, 0), 0))],
    out_specs=pl.BlockSpec((1, BLK, 3), lambda i: (0, i, 0)),
    out_shape=jax.ShapeDtypeStruct((1, NV, 3), jnp.float32),
)


def kernel(verts, edges, subdivided_faces, W1, b1, W2, b2, W3, b3):
  srcp, dstp = _tcprep(edges)
  src2 = srcp.reshape(NCH, CH)
  dst2 = dstp.reshape(NCH, CH)
  verts8 = jnp.pad(verts, ((0, NP - N), (0, 5)))
  w3p = jnp.pad(W3, ((0, 0), (0, 5)))
  w3a = w3p[:16]
  w3b = w3p[16:]
  b3p = jnp.pad(b3, (0, 5))
  z1 = jnp.zeros((NP,), jnp.float32)
  z8 = jnp.zeros((NP, 8), jnp.float32)
  z16 = jnp.zeros((NP, 16), jnp.float32)

  degf = _sc_degree(dst2, z1).reshape(2 * NP)
  dis, y1 = _tc0(degf, degf, verts8)
  acc1 = _scatter8(src2, dst2, y1, z8)
  y2a, y2b = _tc1(acc1, acc1, y1, dis, W1, b1, W2)
  acc2a = _scatter16(src2, dst2, y2a, z16)
  acc2b = _scatter16(src2, dst2, y2b, z16)
  y3p = _tc2(acc2a, acc2a, acc2b, acc2b, y2a, y2b, dis, b2, w3a, w3b)
  acc3 = _scatter8(src2, dst2, y3p, z8)
  v8, vh = _tc3(acc3, acc3, y3p, dis, verts8, b3p)
  gs = _sc_gatherrows(src2, vh)
  gd = _sc_gatherrows(dst2, vh)
  new_verts = _tcasm(v8, gs, gd)
  new_faces = subdivided_faces[None]
  return new_verts, new_faces
